# baseline (speedup 1.0000x reference)
import functools

import jax
import jax.numpy as jnp
from jax import lax
from jax.experimental import pallas as pl
from jax.experimental.pallas import tpu as pltpu
from jax.experimental.pallas import tpu_sc as plsc

EMBED_DIM = 128
NUM_ROWS = 4
BATCH = 16384

_info = plsc.get_sparse_core_info()
_NC = _info.num_cores
_NS = _info.num_subcores
_NW = _NC * _NS
_BPW = BATCH // _NW
_CHUNKS = (32, 96, 128, 128, 128)
assert sum(_CHUNKS) == _BPW
_STARTS = tuple(sum(_CHUNKS[:i]) for i in range(len(_CHUNKS)))

_mesh = plsc.VectorSubcoreMesh(core_axis_name="c", subcore_axis_name="s")


@functools.partial(
    pl.kernel,
    mesh=_mesh,
    out_type=jax.ShapeDtypeStruct((BATCH, EMBED_DIM), jnp.float32),
    scratch_types=[
        pltpu.VMEM((_BPW,), jnp.int32),
        pltpu.VMEM((_BPW, EMBED_DIM), jnp.float32),
        pltpu.VMEM_SHARED((NUM_ROWS, EMBED_DIM), jnp.float32),
        pltpu.SemaphoreType.DMA,
        pltpu.SemaphoreType.DMA,
        pltpu.SemaphoreType.DMA,
    ],
)
def _gather_kernel(ids_hbm, table_hbm, out_hbm, idx_v, rows_v, table_sh,
                   gsem, wsem, tsem):
    sid = lax.axis_index("s")
    cid = lax.axis_index("c")
    wid = sid * _NC + cid
    base = wid * _BPW

    @pl.when(sid == 0)
    def _():
        pltpu.async_copy(table_hbm, table_sh, tsem)

    pltpu.sync_copy(ids_hbm.at[pl.ds(base, _BPW)], idx_v)

    @pl.when(sid == 0)
    def _():
        pltpu.make_async_copy(table_hbm, table_sh, tsem).wait()

    plsc.subcore_barrier()

    for start, size in zip(_STARTS, _CHUNKS):
        pltpu.async_copy(
            table_sh.at[idx_v.at[pl.ds(start, size)]],
            rows_v.at[pl.ds(start, size)],
            gsem,
        )
    for start, size in zip(_STARTS, _CHUNKS):
        pltpu.make_async_copy(
            table_sh.at[idx_v.at[pl.ds(start, size)]],
            rows_v.at[pl.ds(start, size)],
            gsem,
        ).wait()
        pltpu.async_copy(
            rows_v.at[pl.ds(start, size)],
            out_hbm.at[pl.ds(base + start, size)],
            wsem,
        )
    for start, size in zip(_STARTS, _CHUNKS):
        pltpu.make_async_copy(
            rows_v.at[pl.ds(start, size)],
            out_hbm.at[pl.ds(base + start, size)],
            wsem,
        ).wait()


def kernel(archetype_ids, table):
    ids = archetype_ids.astype(jnp.int32)
    return _gather_kernel(ids, table)
